# baseline (speedup 1.0000x reference)
import functools

import jax
import jax.numpy as jnp
from jax import lax
from jax.experimental import pallas as pl
from jax.experimental.pallas import tpu as pltpu
from jax.experimental.pallas import tpu_sc as plsc

N = 10000
D = 128
ALPHA = 0.2

NC, NS, L = 2, 16, 16
NW = NC * NS
CH = 64
T = 320
TP = 40
NB = 4
RACC = N + 112
RPT = RACC // NS


_sc_mesh = plsc.VectorSubcoreMesh(core_axis_name="c", subcore_axis_name="s")


def _run_side(x_hbm, src_hbm, dst_hbm, s, src_v, dst_v, bufs, acc, sems):
    for ph in range(T // TP):
        pltpu.sync_copy(src_hbm.at[s, pl.ds(ph * TP, TP)], src_v)
        pltpu.sync_copy(dst_hbm.at[s, pl.ds(ph * TP, TP)], dst_v)

        for b in range(NB):
            pltpu.async_copy(x_hbm.at[src_v.at[b]], bufs[b], sems[b])

        def _step(t, carry):
            for b in range(NB):
                k = NB * t + b
                pltpu.make_async_copy(x_hbm.at[src_v.at[k]],
                                      bufs[b], sems[b]).wait()
                pltpu.sync_copy(bufs[b], acc.at[dst_v.at[k]], add=True)

                @pl.when(k + NB < TP)
                def _():
                    pltpu.async_copy(x_hbm.at[src_v.at[k + NB]],
                                     bufs[b], sems[b])
            return carry

        lax.fori_loop(0, TP // NB, _step, 0)


@functools.partial(
    pl.kernel,
    out_type=jax.ShapeDtypeStruct((2, RACC, D), jnp.float32),
    mesh=_sc_mesh,
    scratch_types=[
        pltpu.VMEM((TP, CH), jnp.int32),
        pltpu.VMEM((TP, CH), jnp.int32),
        [pltpu.VMEM((CH, D), jnp.float32) for _ in range(NB)],
        pltpu.VMEM_SHARED((RACC, D), jnp.float32),
        [pltpu.SemaphoreType.DMA for _ in range(NB)],
    ],
)
def _spmm2_sc(xa_hbm, xb_hbm, srca_hbm, dsta_hbm, srcb_hbm, dstb_hbm, out_hbm,
              src_v, dst_v, bufs, acc, sems):
    c = lax.axis_index("c")
    s = lax.axis_index("s")
    base = s * RPT

    zero = jnp.zeros((L,), jnp.float32)

    def _zrow(r, carry):
        for g in range(D // L):
            bufs[0][r, pl.ds(g * L, L)] = zero
        return carry

    lax.fori_loop(0, CH, _zrow, 0)
    nfull = RPT // CH
    for z in range(nfull):
        pltpu.sync_copy(bufs[0], acc.at[pl.ds(base + z * CH, CH)])
    rem = RPT - nfull * CH
    if rem:
        pltpu.sync_copy(bufs[0].at[pl.ds(0, rem)],
                        acc.at[pl.ds(base + nfull * CH, rem)])
    plsc.subcore_barrier()

    @pl.when(c == 0)
    def _():
        _run_side(xa_hbm, srca_hbm, dsta_hbm, s, src_v, dst_v, bufs, acc, sems)

    @pl.when(c == 1)
    def _():
        _run_side(xb_hbm, srcb_hbm, dstb_hbm, s, src_v, dst_v, bufs, acc, sems)

    plsc.subcore_barrier()

    pltpu.sync_copy(acc.at[pl.ds(base, RPT)], out_hbm.at[c, pl.ds(base, RPT)])


def _prep_idx(adj):
    e = adj.shape[1]
    npad = NS * T * CH - e
    ar = jnp.arange(npad, dtype=jnp.int32)
    src = jnp.concatenate([adj[1].astype(jnp.int32), ar % N])
    dst = jnp.concatenate([adj[0].astype(jnp.int32), N + ar % (RACC - N)])
    return src.reshape(NS, T, CH), dst.reshape(NS, T, CH)


BR = 1000


def _mm_body(x_ref, w_ref, o_ref):
    o_ref[...] = jnp.dot(x_ref[...], w_ref[...],
                         preferred_element_type=jnp.float32)


def _mm(x, w):
    return pl.pallas_call(
        _mm_body,
        grid=(N // BR,),
        in_specs=[pl.BlockSpec((BR, D), lambda i: (i, 0)),
                  pl.BlockSpec((D, D), lambda i: (0, 0))],
        out_specs=pl.BlockSpec((BR, D), lambda i: (i, 0)),
        out_shape=jax.ShapeDtypeStruct((N, D), jnp.float32),
    )(x, w)


def _combine_body(p_ref, b_ref, w_ref, o_ref):
    h = p_ref[0] + b_ref[...]
    h = jnp.where(h >= 0, h, ALPHA * h)
    o_ref[...] = jnp.dot(h, w_ref[...], preferred_element_type=jnp.float32)


def _combine_mm(p, plane, b, w):
    return pl.pallas_call(
        _combine_body,
        grid=(N // BR,),
        in_specs=[pl.BlockSpec((1, BR, D), lambda i: (plane, i, 0)),
                  pl.BlockSpec((1, D), lambda i: (0, 0)),
                  pl.BlockSpec((D, D), lambda i: (0, 0))],
        out_specs=pl.BlockSpec((BR, D), lambda i: (i, 0)),
        out_shape=jax.ShapeDtypeStruct((N, D), jnp.float32),
    )(p, b.reshape(1, D), w)


def _final_body(p_ref, b_ref, x2_ref, wa_ref, wb_ref, b2_ref, o_ref):
    h = p_ref[0] + b_ref[...]
    h = jnp.where(h >= 0, h, ALPHA * h)
    y = (jnp.dot(h, wa_ref[...], preferred_element_type=jnp.float32)
         + jnp.dot(x2_ref[...], wb_ref[...], preferred_element_type=jnp.float32)
         + b2_ref[...])
    o_ref[...] = jnp.maximum(y, 0.0)


def _final(p, plane, b, x2, wa, wb, b2):
    return pl.pallas_call(
        _final_body,
        grid=(N // BR,),
        in_specs=[pl.BlockSpec((1, BR, D), lambda i: (plane, i, 0)),
                  pl.BlockSpec((1, D), lambda i: (0, 0)),
                  pl.BlockSpec((BR, D), lambda i: (i, 0)),
                  pl.BlockSpec((D, D), lambda i: (0, 0)),
                  pl.BlockSpec((D, D), lambda i: (0, 0)),
                  pl.BlockSpec((1, D), lambda i: (0, 0))],
        out_specs=pl.BlockSpec((BR, D), lambda i: (i, 0)),
        out_shape=jax.ShapeDtypeStruct((N, D), jnp.float32),
    )(p, b.reshape(1, D), x2, wa, wb, b2.reshape(1, D))


def kernel(ufea, vfea, UV_adj, VU_adj, gc1_W, gc1_b, gc2_W, gc2_b,
           gc3_W, gc3_b, gc4_W, gc4_b, uu_W, uu_b, iu_W, iu_b):
    uv_src, uv_dst = _prep_idx(UV_adj)
    vu_src, vu_dst = _prep_idx(VU_adj)

    s1 = _mm(ufea, gc1_W)
    s2 = _mm(vfea, gc2_W)
    pa = _spmm2_sc(s1, s2, vu_src, vu_dst, uv_src, uv_dst)
    s3 = _combine_mm(pa, 0, gc1_b, gc3_W)
    s4 = _combine_mm(pa, 1, gc2_b, gc4_W)
    pb = _spmm2_sc(s3, s4, uv_src, uv_dst, vu_src, vu_dst)
    user = _final(pb, 0, gc3_b, ufea, uu_W[:D], uu_W[D:], uu_b)
    item = _final(pb, 1, gc4_b, vfea, iu_W[:D], iu_W[D:], iu_b)
    return user, item


# speedup vs baseline: 1.0160x; 1.0160x over previous
import functools

import jax
import jax.numpy as jnp
from jax import lax
from jax.experimental import pallas as pl
from jax.experimental.pallas import tpu as pltpu
from jax.experimental.pallas import tpu_sc as plsc

N = 10000
D = 128
ALPHA = 0.2

NC, NS, L = 2, 16, 16
NW = NC * NS
CH = 64
T = 160
TP = 40
NB = 4
RACC = N + 112
RPT = RACC // NS


_sc_mesh = plsc.VectorSubcoreMesh(core_axis_name="c", subcore_axis_name="s")


def _run_side(x_hbm, src_hbm, dst_hbm, wid, src_v, dst_v, bufs, acc, sems):
    for ph in range(T // TP):
        pltpu.sync_copy(src_hbm.at[wid, pl.ds(ph * TP, TP)], src_v)
        pltpu.sync_copy(dst_hbm.at[wid, pl.ds(ph * TP, TP)], dst_v)

        for b in range(NB):
            pltpu.async_copy(x_hbm.at[src_v.at[b]], bufs[b], sems[b])

        def _step(t, carry):
            for b in range(NB):
                k = NB * t + b
                pltpu.make_async_copy(x_hbm.at[src_v.at[k]],
                                      bufs[b], sems[b]).wait()
                pltpu.sync_copy(bufs[b], acc.at[dst_v.at[k]], add=True)

                @pl.when(k + NB < TP)
                def _():
                    pltpu.async_copy(x_hbm.at[src_v.at[k + NB]],
                                     bufs[b], sems[b])
            return carry

        lax.fori_loop(0, TP // NB, _step, 0)


@functools.partial(
    pl.kernel,
    out_type=jax.ShapeDtypeStruct((2, RACC, D), jnp.float32),
    mesh=_sc_mesh,
    scratch_types=[
        pltpu.VMEM((TP, CH), jnp.int32),
        pltpu.VMEM((TP, CH), jnp.int32),
        [pltpu.VMEM((CH, D), jnp.float32) for _ in range(NB)],
        pltpu.VMEM_SHARED((RACC, D), jnp.float32),
        [pltpu.SemaphoreType.DMA for _ in range(NB)],
    ],
)
def _spmm_sc(x_hbm, src_hbm, dst_hbm, out_hbm, src_v, dst_v, bufs, acc, sems):
    c = lax.axis_index("c")
    s = lax.axis_index("s")
    wid = s * NC + c
    base = s * RPT

    zero = jnp.zeros((L,), jnp.float32)

    def _zrow(r, carry):
        for g in range(D // L):
            bufs[0][r, pl.ds(g * L, L)] = zero
        return carry

    lax.fori_loop(0, CH, _zrow, 0)
    nfull = RPT // CH
    for z in range(nfull):
        pltpu.sync_copy(bufs[0], acc.at[pl.ds(base + z * CH, CH)])
    rem = RPT - nfull * CH
    if rem:
        pltpu.sync_copy(bufs[0].at[pl.ds(0, rem)],
                        acc.at[pl.ds(base + nfull * CH, rem)])
    plsc.subcore_barrier()

    _run_side(x_hbm, src_hbm, dst_hbm, wid, src_v, dst_v, bufs, acc, sems)

    plsc.subcore_barrier()

    pltpu.sync_copy(acc.at[pl.ds(base, RPT)], out_hbm.at[c, pl.ds(base, RPT)])


def _prep_idx(adj):
    e = adj.shape[1]
    npad = NW * T * CH - e
    ar = jnp.arange(npad, dtype=jnp.int32)
    src = jnp.concatenate([adj[1].astype(jnp.int32), ar % N])
    dst = jnp.concatenate([adj[0].astype(jnp.int32), N + ar % (RACC - N)])
    return src.reshape(NW, T, CH), dst.reshape(NW, T, CH)


BR = 1000


def _bf16_dot(x, w):
    return jnp.dot(x.astype(jnp.bfloat16), w.astype(jnp.bfloat16),
                   preferred_element_type=jnp.float32)


def _mm_body(x_ref, w_ref, o_ref):
    o_ref[...] = _bf16_dot(x_ref[...], w_ref[...])


def _mm(x, w):
    return pl.pallas_call(
        _mm_body,
        grid=(N // BR,),
        in_specs=[pl.BlockSpec((BR, D), lambda i: (i, 0)),
                  pl.BlockSpec((D, D), lambda i: (0, 0))],
        out_specs=pl.BlockSpec((BR, D), lambda i: (i, 0)),
        out_shape=jax.ShapeDtypeStruct((N, D), jnp.float32),
    )(x, w)


def _combine_body(p0_ref, p1_ref, b_ref, w_ref, o_ref):
    h = p0_ref[0] + p1_ref[0] + b_ref[...]
    h = jnp.where(h >= 0, h, ALPHA * h)
    o_ref[...] = _bf16_dot(h, w_ref[...])


def _combine_mm(p, b, w):
    return pl.pallas_call(
        _combine_body,
        grid=(N // BR,),
        in_specs=[pl.BlockSpec((1, BR, D), lambda i: (0, i, 0)),
                  pl.BlockSpec((1, BR, D), lambda i: (1, i, 0)),
                  pl.BlockSpec((1, D), lambda i: (0, 0)),
                  pl.BlockSpec((D, D), lambda i: (0, 0))],
        out_specs=pl.BlockSpec((BR, D), lambda i: (i, 0)),
        out_shape=jax.ShapeDtypeStruct((N, D), jnp.float32),
    )(p, p, b.reshape(1, D), w)


def _final_body(p0_ref, p1_ref, b_ref, x2_ref, wa_ref, wb_ref, b2_ref, o_ref):
    h = p0_ref[0] + p1_ref[0] + b_ref[...]
    h = jnp.where(h >= 0, h, ALPHA * h)
    y = (_bf16_dot(h, wa_ref[...]) + _bf16_dot(x2_ref[...], wb_ref[...])
         + b2_ref[...])
    o_ref[...] = jnp.maximum(y, 0.0)


def _final(p, b, x2, wa, wb, b2):
    return pl.pallas_call(
        _final_body,
        grid=(N // BR,),
        in_specs=[pl.BlockSpec((1, BR, D), lambda i: (0, i, 0)),
                  pl.BlockSpec((1, BR, D), lambda i: (1, i, 0)),
                  pl.BlockSpec((1, D), lambda i: (0, 0)),
                  pl.BlockSpec((BR, D), lambda i: (i, 0)),
                  pl.BlockSpec((D, D), lambda i: (0, 0)),
                  pl.BlockSpec((D, D), lambda i: (0, 0)),
                  pl.BlockSpec((1, D), lambda i: (0, 0))],
        out_specs=pl.BlockSpec((BR, D), lambda i: (i, 0)),
        out_shape=jax.ShapeDtypeStruct((N, D), jnp.float32),
    )(p, p, b.reshape(1, D), x2, wa, wb, b2.reshape(1, D))


def kernel(ufea, vfea, UV_adj, VU_adj, gc1_W, gc1_b, gc2_W, gc2_b,
           gc3_W, gc3_b, gc4_W, gc4_b, uu_W, uu_b, iu_W, iu_b):
    uv_src, uv_dst = _prep_idx(UV_adj)
    vu_src, vu_dst = _prep_idx(VU_adj)

    s1 = _mm(ufea, gc1_W)
    s2 = _mm(vfea, gc2_W)
    p1 = _spmm_sc(s1, vu_src, vu_dst)
    p2 = _spmm_sc(s2, uv_src, uv_dst)
    s3 = _combine_mm(p1, gc1_b, gc3_W)
    s4 = _combine_mm(p2, gc2_b, gc4_W)
    p3 = _spmm_sc(s3, uv_src, uv_dst)
    p4 = _spmm_sc(s4, vu_src, vu_dst)
    user = _final(p3, gc3_b, ufea, uu_W[:D], uu_W[D:], uu_b)
    item = _final(p4, gc4_b, vfea, iu_W[:D], iu_W[D:], iu_b)
    return user, item
